# baseline (speedup 1.0000x reference)
import jax
import jax.numpy as jnp
from jax import lax
from jax.experimental import pallas as pl
from jax.experimental.pallas import tpu as pltpu
from jax.experimental.pallas import tpu_sc as plsc

TIME_DIM = 64
HOUR_SIZE = 24
DAY_SIZE = 32
MONTH_SIZE = 13
N = 16384
TAB_ROWS = HOUR_SIZE + DAY_SIZE + MONTH_SIZE

NUM_CORES = 2
NUM_SUBCORES = 16
LANES = 16
NW = NUM_CORES * NUM_SUBCORES
B_PER_W = N // NW
N_GROUPS = B_PER_W // LANES

_FIELDS = ((2, 0, HOUR_SIZE), (1, HOUR_SIZE, DAY_SIZE),
           (0, HOUR_SIZE + DAY_SIZE, MONTH_SIZE))


def _body(x_hbm, tab_hbm, out_hbm, tab_v, x_v, idx_v, out_v, sem):
    wid = lax.axis_index("s") * NUM_CORES + lax.axis_index("c")
    base = wid * B_PER_W

    tab_cp = pltpu.async_copy(tab_hbm, tab_v, sem)
    x_cps = [pltpu.async_copy(x_hbm.at[pl.ds(c * N + base, B_PER_W)],
                              x_v.at[pl.ds(c * B_PER_W, B_PER_W)], sem)
             for c in range(3)]
    for cp in x_cps:
        cp.wait()

    refs = []
    acc = None
    for c, (col, off, size) in enumerate(_FIELDS):
        ref_s = None
        for g in range(N_GROUPS):
            vals = x_v[pl.ds(c * B_PER_W + g * LANES, LANES)]
            idx = ((vals + 0.5) * float(size)).astype(jnp.int32)
            idx = (jnp.clip(idx, 0, size - 1) + off) * TIME_DIM
            idx_v[pl.ds(c * B_PER_W + g * LANES, LANES)] = idx
            if ref_s is None:
                ref_s = idx[0]
                refs.append(ref_s)
            same = idx == ref_s
            acc = same if acc is None else jnp.logical_and(acc, same)
    uniform = plsc.all_reduce_population_count(acc)[0] == LANES
    tab_cp.wait()

    @pl.when(uniform)
    def _fast():
        rows = [tab_v[pl.ds(refs[0] + j * LANES, LANES)]
                + tab_v[pl.ds(refs[1] + j * LANES, LANES)]
                + tab_v[pl.ds(refs[2] + j * LANES, LANES)]
                for j in range(TIME_DIM // LANES)]
        for d in range(TIME_DIM):
            vec = jnp.full((LANES,), rows[d // LANES][d % LANES],
                           dtype=jnp.float32)
            for k in range(128 // LANES):
                out_v[d, pl.ds(k * LANES, LANES)] = vec
        out_cps = [pltpu.async_copy(
            out_v.at[:, pl.ds(0, 128)],
            out_hbm.at[:, pl.ds(base + q * 128, 128)], sem)
            for q in range(B_PER_W // 128)]
        for cp in out_cps:
            cp.wait()

    @pl.when(jnp.logical_not(uniform))
    def _slow():
        def group(g, carry):
            iv0 = idx_v[pl.ds(g * LANES, LANES)]
            iv1 = idx_v[pl.ds(B_PER_W + g * LANES, LANES)]
            iv2 = idx_v[pl.ds(2 * B_PER_W + g * LANES, LANES)]
            for d in range(TIME_DIM):
                col = (plsc.load_gather(tab_v, [iv0 + d])
                       + plsc.load_gather(tab_v, [iv1 + d])
                       + plsc.load_gather(tab_v, [iv2 + d]))
                out_v[d, pl.ds(g * LANES, LANES)] = col
            return carry

        lax.fori_loop(0, N_GROUPS, group, 0)
        pltpu.sync_copy(out_v, out_hbm.at[:, pl.ds(base, B_PER_W)])


@jax.jit
def _lookup(x_flat, tab_flat):
    mesh = plsc.VectorSubcoreMesh(core_axis_name="c", subcore_axis_name="s")
    run = pl.kernel(
        _body,
        out_type=jax.ShapeDtypeStruct((TIME_DIM, N), jnp.float32),
        mesh=mesh,
        scratch_types=[
            pltpu.VMEM((TAB_ROWS * TIME_DIM,), jnp.float32),
            pltpu.VMEM((3 * B_PER_W,), jnp.float32),
            pltpu.VMEM((3 * B_PER_W,), jnp.int32),
            pltpu.VMEM((TIME_DIM, B_PER_W), jnp.float32),
            pltpu.SemaphoreType.DMA,
        ],
        compiler_params=pltpu.CompilerParams(needs_layout_passes=False),
    )
    return run(x_flat, tab_flat)


def kernel(x, hour_embed, day_embed, month_embed):
    tab = jnp.concatenate([hour_embed, day_embed, month_embed], axis=0)
    return _lookup(x.T.reshape(-1), tab.reshape(-1)).T


# speedup vs baseline: 1.0085x; 1.0085x over previous
import jax
import jax.numpy as jnp
from jax import lax
from jax.experimental import pallas as pl
from jax.experimental.pallas import tpu as pltpu
from jax.experimental.pallas import tpu_sc as plsc

TIME_DIM = 64
HOUR_SIZE = 24
DAY_SIZE = 32
MONTH_SIZE = 13
N = 16384
TAB_ROWS = HOUR_SIZE + DAY_SIZE + MONTH_SIZE

NUM_CORES = 2
NUM_SUBCORES = 16
LANES = 16
NW = NUM_CORES * NUM_SUBCORES
B_PER_W = N // NW
N_GROUPS = B_PER_W // LANES

_FIELDS = ((2, 0, HOUR_SIZE), (1, HOUR_SIZE, DAY_SIZE),
           (0, HOUR_SIZE + DAY_SIZE, MONTH_SIZE))


def _body(x_hbm, tab_hbm, out_hbm, tab_v, x_v, idx_v, out_v, slab_v, sem):
    wid = lax.axis_index("s") * NUM_CORES + lax.axis_index("c")
    base = wid * B_PER_W

    tab_cp = pltpu.async_copy(tab_hbm, tab_v, sem)
    x_cps = [pltpu.async_copy(x_hbm.at[pl.ds(c * N + base, B_PER_W)],
                              x_v.at[pl.ds(c * B_PER_W, B_PER_W)], sem)
             for c in range(3)]
    for cp in x_cps:
        cp.wait()

    refs = []
    acc = None
    for c, (col, off, size) in enumerate(_FIELDS):
        ref_s = None
        for g in range(N_GROUPS):
            vals = x_v[pl.ds(c * B_PER_W + g * LANES, LANES)]
            idx = ((vals + 0.5) * float(size)).astype(jnp.int32)
            idx = (jnp.clip(idx, 0, size - 1) + off) * TIME_DIM
            idx_v[pl.ds(c * B_PER_W + g * LANES, LANES)] = idx
            if ref_s is None:
                ref_s = idx[0]
                refs.append(ref_s)
            same = idx == ref_s
            acc = same if acc is None else jnp.logical_and(acc, same)
    uniform = plsc.all_reduce_population_count(acc)[0] == LANES
    tab_cp.wait()

    @pl.when(uniform)
    def _fast():
        rows = [tab_v[pl.ds(refs[0] + j * LANES, LANES)]
                + tab_v[pl.ds(refs[1] + j * LANES, LANES)]
                + tab_v[pl.ds(refs[2] + j * LANES, LANES)]
                for j in range(TIME_DIM // LANES)]
        for d in range(TIME_DIM):
            vec = jnp.full((LANES,), rows[d // LANES][d % LANES],
                           dtype=jnp.float32)
            for k in range(128 // LANES):
                slab_v[d, pl.ds(k * LANES, LANES)] = vec
        out_cps = [pltpu.async_copy(
            slab_v, out_hbm.at[:, pl.ds(base + q * 128, 128)], sem)
            for q in range(B_PER_W // 128)]
        for cp in out_cps:
            cp.wait()

    @pl.when(jnp.logical_not(uniform))
    def _slow():
        def group(g, carry):
            iv0 = idx_v[pl.ds(g * LANES, LANES)]
            iv1 = idx_v[pl.ds(B_PER_W + g * LANES, LANES)]
            iv2 = idx_v[pl.ds(2 * B_PER_W + g * LANES, LANES)]
            for d in range(TIME_DIM):
                col = (plsc.load_gather(tab_v, [iv0 + d])
                       + plsc.load_gather(tab_v, [iv1 + d])
                       + plsc.load_gather(tab_v, [iv2 + d]))
                out_v[d, pl.ds(g * LANES, LANES)] = col
            return carry

        lax.fori_loop(0, N_GROUPS, group, 0)
        pltpu.sync_copy(out_v, out_hbm.at[:, pl.ds(base, B_PER_W)])


@jax.jit
def _lookup(x_flat, tab_flat):
    mesh = plsc.VectorSubcoreMesh(core_axis_name="c", subcore_axis_name="s")
    run = pl.kernel(
        _body,
        out_type=jax.ShapeDtypeStruct((TIME_DIM, N), jnp.float32),
        mesh=mesh,
        scratch_types=[
            pltpu.VMEM((TAB_ROWS * TIME_DIM,), jnp.float32),
            pltpu.VMEM((3 * B_PER_W,), jnp.float32),
            pltpu.VMEM((3 * B_PER_W,), jnp.int32),
            pltpu.VMEM((TIME_DIM, B_PER_W), jnp.float32),
            pltpu.VMEM((TIME_DIM, 128), jnp.float32),
            pltpu.SemaphoreType.DMA,
        ],
        compiler_params=pltpu.CompilerParams(needs_layout_passes=False),
    )
    return run(x_flat, tab_flat)


def kernel(x, hour_embed, day_embed, month_embed):
    tab = jnp.concatenate([hour_embed, day_embed, month_embed], axis=0)
    return _lookup(x.T.reshape(-1), tab.reshape(-1)).T
